# baseline (speedup 1.0000x reference)
import functools

import jax
import jax.numpy as jnp
from jax import lax
from jax.experimental import pallas as pl
from jax.experimental.pallas import tpu as pltpu
from jax.experimental.pallas import tpu_sc as plsc

B, S, D, H, NV, DR = 1, 2048, 1024, 16, 8192, 64
DH = D // H
SB = 256
NSB = S // SB
NEG = -1e30
QSCALE = 0.125


def _code_body(x_ref, wvq_ref, bvq_ref, vkt_ref, idx_ref):
    vq = jnp.dot(x_ref[...], wvq_ref[...],
                 preferred_element_type=jnp.float32) + bvq_ref[...]
    sim = jnp.dot(vq, vkt_ref[...], preferred_element_type=jnp.float32)
    m = jnp.max(sim, axis=1, keepdims=True)
    col = lax.broadcasted_iota(jnp.int32, sim.shape, 1)
    cand = jnp.where(sim == m, col, NV)
    idx_ref[0, 0, :] = jnp.min(cand, axis=1)


def _code_proj(x, Wvq, bvq, v_keys_t):
    full = lambda shape: pl.BlockSpec(shape, lambda i: (0,) * len(shape))
    return pl.pallas_call(
        _code_body,
        grid=(NSB,),
        in_specs=[
            pl.BlockSpec((SB, D), lambda i: (i, 0)),
            full((D, DR)), full((1, DR)),
            full((DR, NV)),
        ],
        out_specs=pl.BlockSpec((1, 1, SB), lambda i: (i, 0, 0)),
        out_shape=jax.ShapeDtypeStruct((NSB, 1, SB), jnp.int32),
    )(x, Wvq, bvq.reshape(1, DR), v_keys_t)


def _qk_body(x_ref, wq_ref, bq_ref, wk_ref, bk_ref, q_ref, k_ref):
    x = x_ref[...].astype(jnp.bfloat16)
    q = jnp.dot(x, wq_ref[...].astype(jnp.bfloat16),
                preferred_element_type=jnp.float32) + bq_ref[...]
    q_ref[...] = (q * QSCALE).astype(jnp.bfloat16)
    k = jnp.dot(x, wk_ref[...].astype(jnp.bfloat16),
                preferred_element_type=jnp.float32) + bk_ref[...]
    k_ref[...] = k.astype(jnp.bfloat16)


def _qk_proj(x, Wq, bq, Wk, bk):
    full = lambda shape: pl.BlockSpec(shape, lambda i: (0,) * len(shape))
    seq = pl.BlockSpec((SB, D), lambda i: (i, 0))
    return pl.pallas_call(
        _qk_body,
        grid=(NSB,),
        in_specs=[seq, full((D, D)), full((1, D)), full((D, D)),
                  full((1, D))],
        out_specs=[seq, seq],
        out_shape=[jax.ShapeDtypeStruct((S, D), jnp.bfloat16),
                   jax.ShapeDtypeStruct((S, D), jnp.bfloat16)],
    )(x, Wq, bq.reshape(1, D), Wk, bk.reshape(1, D))


_NW = 32
_BPW = S // _NW


def _sc_gather(idx, table):
    mesh = plsc.VectorSubcoreMesh(core_axis_name="c", subcore_axis_name="s")

    @functools.partial(
        pl.kernel, mesh=mesh,
        out_type=jax.ShapeDtypeStruct((S, D), jnp.float32),
        scratch_types=[
            pltpu.VMEM((_BPW,), jnp.int32),
            pltpu.VMEM((_BPW, D), jnp.float32),
            pltpu.SemaphoreType.DMA,
        ],
    )
    def k(idx_hbm, table_hbm, out_hbm, idx_v, rows_v, sem):
        wid = lax.axis_index("s") * 2 + lax.axis_index("c")
        base = wid * _BPW
        pltpu.sync_copy(idx_hbm.at[pl.ds(base, _BPW)], idx_v)
        pltpu.async_copy(table_hbm.at[idx_v], rows_v, sem).wait()
        pltpu.sync_copy(rows_v, out_hbm.at[pl.ds(base, _BPW)])

    return k(idx, table)


RB = 256
NRB = S // RB


def _attn_body(q_ref, k_ref, x_ref, vs_ref, o_ref):
    vfull = (x_ref[...] * vs_ref[...]).astype(jnp.bfloat16)
    diag_ok = (lax.broadcasted_iota(jnp.int32, (RB, RB), 0)
               >= lax.broadcasted_iota(jnp.int32, (RB, RB), 1))
    for r in range(NRB):
        w = (r + 1) * RB
        rows = slice(r * RB, (r + 1) * RB)
        for j in range(2):
            sl = slice(j * DH, (j + 1) * DH)
            q = q_ref[rows, sl]
            k = k_ref[:w, sl]
            s = lax.dot_general(q, k, (((1,), (1,)), ((), ())),
                                preferred_element_type=jnp.float32)
            sd = jnp.where(diag_ok, s[:, r * RB:], NEG)
            m = jnp.max(sd, axis=1, keepdims=True)
            if r:
                m = jnp.maximum(m, jnp.max(s[:, :r * RB], axis=1,
                                           keepdims=True))
                pp = jnp.exp(s[:, :r * RB] - m)
            pd = jnp.exp(sd - m)
            l = jnp.sum(pd, axis=1, keepdims=True)
            if r:
                l = l + jnp.sum(pp, axis=1, keepdims=True)
            v = vfull[:w, sl]
            acc = jnp.dot(pd.astype(jnp.bfloat16), v[r * RB:],
                          preferred_element_type=jnp.float32)
            if r:
                acc = acc + jnp.dot(pp.astype(jnp.bfloat16), v[:r * RB],
                                    preferred_element_type=jnp.float32)
            o_ref[rows, sl] = (acc / l).astype(jnp.bfloat16)


def _attention(q, k, x, v_sel):
    pair = pl.BlockSpec((S, 2 * DH), lambda h: (0, h))
    return pl.pallas_call(
        _attn_body,
        grid=(H // 2,),
        in_specs=[pair, pair, pair, pair],
        out_specs=pair,
        out_shape=jax.ShapeDtypeStruct((S, D), jnp.bfloat16),
    )(q, k, x, v_sel)


def _out_body(a_ref, wo_ref, bo_ref, o_ref):
    o_ref[...] = jnp.dot(a_ref[...], wo_ref[...].astype(jnp.bfloat16),
                         preferred_element_type=jnp.float32) + bo_ref[...]


def _out_proj(attn, Wo, bo):
    return pl.pallas_call(
        _out_body,
        out_shape=jax.ShapeDtypeStruct((S, D), jnp.float32),
    )(attn, Wo, bo.reshape(1, D))


def kernel(hidden_states, Wq, bq, Wk, bk, Wvq, bvq, v_keys, v_embed, Wo, bo):
    x = hidden_states.reshape(S, D)
    idx3 = _code_proj(x, Wvq, bvq, v_keys.T)
    v_sel = _sc_gather(idx3.reshape(S), v_embed)
    q, k = _qk_proj(x, Wq, bq, Wk, bk)
    attn = _attention(q, k, x, v_sel)
    out = _out_proj(attn, Wo, bo)
    return out.reshape(B, S, D)


# speedup vs baseline: 1.1278x; 1.1278x over previous
import functools

import jax
import jax.numpy as jnp
from jax import lax
from jax.experimental import pallas as pl
from jax.experimental.pallas import tpu as pltpu
from jax.experimental.pallas import tpu_sc as plsc

B, S, D, H, NV, DR = 1, 2048, 1024, 16, 8192, 64
DH = D // H
SB = 256
NSB = S // SB
NEG = -1e30
QSCALE = 0.125


def _code_body(x_ref, wvq_ref, bvq_ref, vkt_ref, idx_ref):
    vq = jnp.dot(x_ref[...], wvq_ref[...],
                 preferred_element_type=jnp.float32) + bvq_ref[...]
    sim = jnp.dot(vq, vkt_ref[...], preferred_element_type=jnp.float32)
    m = jnp.max(sim, axis=1, keepdims=True)
    col = lax.broadcasted_iota(jnp.int32, sim.shape, 1)
    cand = jnp.where(sim == m, col, NV)
    idx_ref[0, 0, :] = jnp.min(cand, axis=1)


def _code_proj(x, Wvq, bvq, v_keys_t):
    full = lambda shape: pl.BlockSpec(shape, lambda i: (0,) * len(shape))
    return pl.pallas_call(
        _code_body,
        grid=(NSB,),
        in_specs=[
            pl.BlockSpec((SB, D), lambda i: (i, 0)),
            full((D, DR)), full((1, DR)),
            full((DR, NV)),
        ],
        out_specs=pl.BlockSpec((1, 1, SB), lambda i: (i, 0, 0)),
        out_shape=jax.ShapeDtypeStruct((NSB, 1, SB), jnp.int32),
    )(x, Wvq, bvq.reshape(1, DR), v_keys_t)


def _qk_body(x_ref, wq_ref, bq_ref, wk_ref, bk_ref, q_ref, k_ref):
    x = x_ref[...].astype(jnp.bfloat16)
    q = jnp.dot(x, wq_ref[...].astype(jnp.bfloat16),
                preferred_element_type=jnp.float32) + bq_ref[...]
    q_ref[...] = (q * QSCALE).astype(jnp.bfloat16)
    k = jnp.dot(x, wk_ref[...].astype(jnp.bfloat16),
                preferred_element_type=jnp.float32) + bk_ref[...]
    k_ref[...] = k.astype(jnp.bfloat16)


def _qk_proj(x, Wq, bq, Wk, bk):
    full = lambda shape: pl.BlockSpec(shape, lambda i: (0,) * len(shape))
    seq = pl.BlockSpec((SB, D), lambda i: (i, 0))
    return pl.pallas_call(
        _qk_body,
        grid=(NSB,),
        in_specs=[seq, full((D, D)), full((1, D)), full((D, D)),
                  full((1, D))],
        out_specs=[seq, seq],
        out_shape=[jax.ShapeDtypeStruct((S, D), jnp.bfloat16),
                   jax.ShapeDtypeStruct((S, D), jnp.bfloat16)],
    )(x, Wq, bq.reshape(1, D), Wk, bk.reshape(1, D))


_NW = 32
_BPW = S // _NW


def _sc_gather(idx, table):
    mesh = plsc.VectorSubcoreMesh(core_axis_name="c", subcore_axis_name="s")

    @functools.partial(
        pl.kernel, mesh=mesh,
        out_type=jax.ShapeDtypeStruct((S, D), jnp.float32),
        scratch_types=[
            pltpu.VMEM((_BPW,), jnp.int32),
            pltpu.VMEM((_BPW, D), jnp.float32),
            pltpu.SemaphoreType.DMA,
        ],
    )
    def k(idx_hbm, table_hbm, out_hbm, idx_v, rows_v, sem):
        wid = lax.axis_index("s") * 2 + lax.axis_index("c")
        base = wid * _BPW
        pltpu.sync_copy(idx_hbm.at[pl.ds(base, _BPW)], idx_v)
        pltpu.async_copy(table_hbm.at[idx_v], rows_v, sem).wait()
        pltpu.sync_copy(rows_v, out_hbm.at[pl.ds(base, _BPW)])

    return k(idx, table)


RB = 512
NRB = S // RB


def _attn_body(q_ref, k_ref, x_ref, vs_ref, o_ref):
    vfull = (x_ref[...] * vs_ref[...]).astype(jnp.bfloat16)
    diag_ok = (lax.broadcasted_iota(jnp.int32, (RB, RB), 0)
               >= lax.broadcasted_iota(jnp.int32, (RB, RB), 1))
    for r in range(NRB):
        w = (r + 1) * RB
        rows = slice(r * RB, (r + 1) * RB)
        for j in range(4):
            sl = slice(j * DH, (j + 1) * DH)
            q = q_ref[rows, sl]
            k = k_ref[:w, sl]
            s = lax.dot_general(q, k, (((1,), (1,)), ((), ())),
                                preferred_element_type=jnp.float32)
            sd = jnp.where(diag_ok, s[:, r * RB:], NEG)
            m = jnp.max(sd, axis=1, keepdims=True)
            if r:
                m = jnp.maximum(m, jnp.max(s[:, :r * RB], axis=1,
                                           keepdims=True))
                pp = jnp.exp(s[:, :r * RB] - m)
            pd = jnp.exp(sd - m)
            l = jnp.sum(pd, axis=1, keepdims=True)
            if r:
                l = l + jnp.sum(pp, axis=1, keepdims=True)
            v = vfull[:w, sl]
            acc = jnp.dot(pd.astype(jnp.bfloat16), v[r * RB:],
                          preferred_element_type=jnp.float32)
            if r:
                acc = acc + jnp.dot(pp.astype(jnp.bfloat16), v[:r * RB],
                                    preferred_element_type=jnp.float32)
            o_ref[rows, sl] = (acc / l).astype(jnp.bfloat16)


def _attention(q, k, x, v_sel):
    pair = pl.BlockSpec((S, 4 * DH), lambda h: (0, h))
    return pl.pallas_call(
        _attn_body,
        grid=(H // 4,),
        in_specs=[pair, pair, pair, pair],
        out_specs=pair,
        out_shape=jax.ShapeDtypeStruct((S, D), jnp.bfloat16),
    )(q, k, x, v_sel)


def _out_body(a_ref, wo_ref, bo_ref, o_ref):
    o_ref[...] = jnp.dot(a_ref[...], wo_ref[...].astype(jnp.bfloat16),
                         preferred_element_type=jnp.float32) + bo_ref[...]


def _out_proj(attn, Wo, bo):
    return pl.pallas_call(
        _out_body,
        out_shape=jax.ShapeDtypeStruct((S, D), jnp.float32),
    )(attn, Wo, bo.reshape(1, D))


def kernel(hidden_states, Wq, bq, Wk, bk, Wvq, bvq, v_keys, v_embed, Wo, bo):
    x = hidden_states.reshape(S, D)
    idx3 = _code_proj(x, Wvq, bvq, v_keys.T)
    v_sel = _sc_gather(idx3.reshape(S), v_embed)
    q, k = _qk_proj(x, Wq, bq, Wk, bk)
    attn = _attention(q, k, x, v_sel)
    out = _out_proj(attn, Wo, bo)
    return out.reshape(B, S, D)
